# baseline (speedup 1.0000x reference)
import functools

import jax
import jax.numpy as jnp
from jax import lax
from jax.experimental import pallas as pl
from jax.experimental.pallas import tpu as pltpu
from jax.experimental.pallas import tpu_sc as plsc

_B, _H, _W, _C = 32, 512, 512, 3
_N = _B * _H * _W * _C
_NW = 32
_CHUNK = _N // _NW
_BLK = 49152
_NBLK = _CHUNK // _BLK


def _sc_body(in_hbm, out_hbm, in_v, out_v):
    cid = lax.axis_index("c")
    sid = lax.axis_index("s")
    wid = sid * 2 + cid

    lane = lax.iota(jnp.int32, 16)
    gidx = []
    for t in range(3):
        delta = 2 - 2 * ((lane + t) % 3)
        gidx.append(16 * t + lane + delta)

    def blk_body(b, carry):
        start = wid * _CHUNK + b * _BLK
        pltpu.sync_copy(in_hbm.at[pl.ds(start, _BLK)], in_v)

        def grp(j, c):
            o = 48 * j
            for t in range(3):
                vals = plsc.load_gather(in_v, [o + gidx[t]])
                out_v[pl.ds(o + 16 * t, 16)] = vals
            return c

        lax.fori_loop(0, _BLK // 48, grp, 0)
        pltpu.sync_copy(out_v, out_hbm.at[pl.ds(start, _BLK)])
        return carry

    lax.fori_loop(0, _NBLK, blk_body, 0)


def kernel(inputs):
    x = inputs.reshape(_N)
    mesh = plsc.VectorSubcoreMesh(core_axis_name="c", subcore_axis_name="s")
    run = functools.partial(
        pl.kernel,
        mesh=mesh,
        out_type=jax.ShapeDtypeStruct((_N,), jnp.float32),
        scratch_types=[
            pltpu.VMEM((_BLK,), jnp.float32),
            pltpu.VMEM((_BLK,), jnp.float32),
        ],
        compiler_params=pltpu.CompilerParams(needs_layout_passes=False),
    )(_sc_body)
    out = run(x)
    return out.reshape(_B, _H, _W, _C)
